# Initial kernel scaffold; baseline (speedup 1.0000x reference)
import functools

import jax
import jax.numpy as jnp
from jax import lax
from jax.experimental import pallas as pl
from jax.experimental.pallas import tpu as pltpu
from jax.experimental.pallas import tpu_sc as plsc

N = 10000
M = 32
ATOM = 128
NBR = 16
GROUPS = 8
E = N * M
ROWS = E // GROUPS
NW = 32
E_PER_W = E // NW
VECS_PER_W = E_PER_W // 16


def _edge_scalar_body(r_ref, w1b_ref, b1t_ref, k2_ref, b2s_ref, s_ref):
    x = r_ref[...]
    h = jnp.dot(x, w1b_ref[...], preferred_element_type=jnp.float32)
    h = h + b1t_ref[...]
    h = jnp.maximum(h, 0.0) + jnp.log1p(jnp.exp(-jnp.abs(h)))
    s = jnp.dot(h, k2_ref[...], preferred_element_type=jnp.float32)
    s_ref[...] = s + b2s_ref[0]


def _edge_scalars(r8, w1b, b1t, k2, b2s):
    BA = 4000
    grid = ROWS // BA
    return pl.pallas_call(
        _edge_scalar_body,
        grid=(grid,),
        in_specs=[
            pl.BlockSpec((BA, 128), lambda g: (g, 0)),
            pl.BlockSpec((128, 128), lambda g: (0, 0)),
            pl.BlockSpec((1, 128), lambda g: (0, 0)),
            pl.BlockSpec((128, GROUPS), lambda g: (0, 0)),
            pl.BlockSpec(memory_space=pltpu.SMEM),
        ],
        out_specs=pl.BlockSpec((BA, GROUPS), lambda g: (g, 0)),
        out_shape=jax.ShapeDtypeStruct((ROWS, GROUPS), jnp.float32),
    )(r8, w1b, b1t, k2, b2s)


def _sc_scatter(flat_idx, s_flat):
    mesh = plsc.VectorSubcoreMesh(core_axis_name="c", subcore_axis_name="s")

    @functools.partial(
        pl.kernel,
        mesh=mesh,
        out_type=[
            jax.ShapeDtypeStruct((NW, N), jnp.float32),
            jax.ShapeDtypeStruct((NW, N), jnp.float32),
        ],
        scratch_types=[
            pltpu.VMEM((E_PER_W,), jnp.int32),
            pltpu.VMEM((E_PER_W,), jnp.float32),
            pltpu.VMEM((N,), jnp.float32),
            pltpu.VMEM((N,), jnp.float32),
        ],
    )
    def scatter_kernel(idx_hbm, s_hbm, osum_hbm, ocnt_hbm,
                       idx_v, s_v, sum_v, cnt_v):
        wid = lax.axis_index("s") * 2 + lax.axis_index("c")
        base = wid * E_PER_W
        pltpu.sync_copy(idx_hbm.at[pl.ds(base, E_PER_W)], idx_v)
        pltpu.sync_copy(s_hbm.at[pl.ds(base, E_PER_W)], s_v)

        zeros = jnp.zeros((16,), jnp.float32)

        def zero_body(i, _):
            sum_v[pl.ds(i * 16, 16)] = zeros
            cnt_v[pl.ds(i * 16, 16)] = zeros
            return ()

        lax.fori_loop(0, N // 16, zero_body, ())

        ones = jnp.ones((16,), jnp.float32)

        def acc_body(i, _):
            idx16 = idx_v[pl.ds(i * 16, 16)]
            s16 = s_v[pl.ds(i * 16, 16)]
            plsc.addupdate_scatter(sum_v, [idx16], s16)
            plsc.addupdate_scatter(cnt_v, [idx16], ones)
            return ()

        lax.fori_loop(0, VECS_PER_W, acc_body, ())

        pltpu.sync_copy(sum_v, osum_hbm.at[wid])
        pltpu.sync_copy(cnt_v, ocnt_hbm.at[wid])

    return scatter_kernel(flat_idx, s_flat)


def _finish_body(sum_ref, cnt_ref, atom_ref, tpw_ref, out_ref):
    ssum = jnp.sum(sum_ref[...], axis=0)
    cnt = jnp.sum(cnt_ref[...], axis=0)
    mean = ssum / jnp.maximum(cnt, 1.0)
    scale = mean * (1.0 / jnp.sqrt(float(ATOM)))
    p = jnp.dot(atom_ref[...], tpw_ref[...], preferred_element_type=jnp.float32)
    out_ref[...] = p * scale[:, None]


def _finish(psum, pcnt, atom_fea, tp_w):
    BC = 1000
    grid = N // BC
    return pl.pallas_call(
        _finish_body,
        grid=(grid,),
        in_specs=[
            pl.BlockSpec((NW, BC), lambda g: (0, g)),
            pl.BlockSpec((NW, BC), lambda g: (0, g)),
            pl.BlockSpec((BC, ATOM), lambda g: (g, 0)),
            pl.BlockSpec((ATOM, ATOM), lambda g: (0, 0)),
        ],
        out_specs=pl.BlockSpec((BC, ATOM), lambda g: (g, 0)),
        out_shape=jax.ShapeDtypeStruct((N, ATOM), jnp.float32),
    )(psum, pcnt, atom_fea, tp_w)


def kernel(atom_fea, nbr_fea, nbr_idx, pos, W1, b1, W2, b2, tp_w):
    del pos
    w1b = jnp.kron(jnp.eye(GROUPS, dtype=jnp.float32), W1)
    b1t = jnp.tile(b1, GROUPS)[None, :]
    k2 = jnp.kron(jnp.eye(GROUPS, dtype=jnp.float32), W2[:, 0:1])
    b2s = jnp.full((1,), b2[0], dtype=jnp.float32)

    r8 = nbr_fea.reshape(ROWS, 128)
    s8 = _edge_scalars(r8, w1b, b1t, k2, b2s)
    s_flat = s8.reshape(E)

    flat_idx = nbr_idx.reshape(E).astype(jnp.int32)
    psum, pcnt = _sc_scatter(flat_idx, s_flat)

    return _finish(psum, pcnt, atom_fea, tp_w)


# speedup vs baseline: 18.9527x; 18.9527x over previous
import functools

import jax
import jax.numpy as jnp
from jax import lax
from jax.experimental import pallas as pl
from jax.experimental.pallas import tpu as pltpu
from jax.experimental.pallas import tpu_sc as plsc

N = 10000
M = 32
ATOM = 128
NBR = 16
GROUPS = 8
E = N * M
ROWS = E // GROUPS
NW = 32
E_PER_W = E // NW
VECS_PER_W = E_PER_W // 16
NPAD = 10240


def _edge_scalar_body(r_ref, w1b_ref, b1t_ref, k2_ref, b2s_ref, s_ref):
    x = r_ref[...]
    h = jnp.dot(x, w1b_ref[...], preferred_element_type=jnp.float32)
    h = h + b1t_ref[...]
    h = jnp.maximum(h, 0.0) + jnp.log1p(jnp.exp(-jnp.abs(h)))
    s = jnp.dot(h, k2_ref[...], preferred_element_type=jnp.float32)
    s_ref[...] = s + b2s_ref[0]


def _edge_scalars(r8, w1b, b1t, k2, b2s):
    BA = 4000
    grid = ROWS // BA
    return pl.pallas_call(
        _edge_scalar_body,
        grid=(grid,),
        in_specs=[
            pl.BlockSpec((BA, 128), lambda g: (g, 0)),
            pl.BlockSpec((128, 128), lambda g: (0, 0)),
            pl.BlockSpec((1, 128), lambda g: (0, 0)),
            pl.BlockSpec((128, GROUPS), lambda g: (0, 0)),
            pl.BlockSpec(memory_space=pltpu.SMEM),
        ],
        out_specs=pl.BlockSpec((BA, GROUPS), lambda g: (g, 0)),
        out_shape=jax.ShapeDtypeStruct((ROWS, GROUPS), jnp.float32),
    )(r8, w1b, b1t, k2, b2s)


def _sc_scatter(flat_idx, s_flat):
    mesh = plsc.VectorSubcoreMesh(core_axis_name="c", subcore_axis_name="s")

    @functools.partial(
        pl.kernel,
        mesh=mesh,
        out_type=[
            jax.ShapeDtypeStruct((NW, NPAD), jnp.float32),
            jax.ShapeDtypeStruct((NW, NPAD), jnp.float32),
        ],
        scratch_types=[
            pltpu.VMEM((E_PER_W,), jnp.int32),
            pltpu.VMEM((E_PER_W,), jnp.float32),
            pltpu.VMEM((NPAD,), jnp.float32),
            pltpu.VMEM((NPAD,), jnp.float32),
        ],
        compiler_params=pltpu.CompilerParams(needs_layout_passes=False),
    )
    def scatter_kernel(idx_hbm, s_hbm, osum_hbm, ocnt_hbm,
                       idx_v, s_v, sum_v, cnt_v):
        wid = lax.axis_index("s") * 2 + lax.axis_index("c")
        base = wid * E_PER_W
        pltpu.sync_copy(idx_hbm.at[pl.ds(base, E_PER_W)], idx_v)
        pltpu.sync_copy(s_hbm.at[pl.ds(base, E_PER_W)], s_v)

        zeros = jnp.zeros((16,), jnp.float32)

        def zero_body(i, _):
            sum_v[pl.ds(i * 16, 16)] = zeros
            cnt_v[pl.ds(i * 16, 16)] = zeros
            return ()

        lax.fori_loop(0, NPAD // 16, zero_body, ())

        ones = jnp.ones((16,), jnp.float32)

        def acc_body(i, _):
            idx16 = idx_v[pl.ds(i * 16, 16)]
            s16 = s_v[pl.ds(i * 16, 16)]
            plsc.addupdate_scatter(sum_v, [idx16], s16)
            plsc.addupdate_scatter(cnt_v, [idx16], ones)
            return ()

        lax.fori_loop(0, VECS_PER_W, acc_body, ())

        pltpu.sync_copy(sum_v, osum_hbm.at[wid])
        pltpu.sync_copy(cnt_v, ocnt_hbm.at[wid])

    return scatter_kernel(flat_idx, s_flat)


def _finish_body(sum_ref, cnt_ref, atom_ref, tpw_ref, out_ref):
    ssum = jnp.sum(sum_ref[...], axis=0)
    cnt = jnp.sum(cnt_ref[...], axis=0)
    mean = ssum / jnp.maximum(cnt, 1.0)
    scale = mean * (1.0 / jnp.sqrt(float(ATOM)))
    p = jnp.dot(atom_ref[...], tpw_ref[...], preferred_element_type=jnp.float32)
    out_ref[...] = p * scale[:, None]


def _finish(psum, pcnt, atom_fea, tp_w):
    BC = 1024
    grid = NPAD // BC
    return pl.pallas_call(
        _finish_body,
        grid=(grid,),
        in_specs=[
            pl.BlockSpec((NW, BC), lambda g: (0, g)),
            pl.BlockSpec((NW, BC), lambda g: (0, g)),
            pl.BlockSpec((BC, ATOM), lambda g: (g, 0)),
            pl.BlockSpec((ATOM, ATOM), lambda g: (0, 0)),
        ],
        out_specs=pl.BlockSpec((BC, ATOM), lambda g: (g, 0)),
        out_shape=jax.ShapeDtypeStruct((N, ATOM), jnp.float32),
    )(psum, pcnt, atom_fea, tp_w)


def kernel(atom_fea, nbr_fea, nbr_idx, pos, W1, b1, W2, b2, tp_w):
    del pos
    w1b = jnp.kron(jnp.eye(GROUPS, dtype=jnp.float32), W1)
    b1t = jnp.tile(b1, GROUPS)[None, :]
    k2 = jnp.kron(jnp.eye(GROUPS, dtype=jnp.float32), W2[:, 0:1])
    b2s = jnp.full((1,), b2[0], dtype=jnp.float32)

    r8 = nbr_fea.reshape(ROWS, 128)
    s8 = _edge_scalars(r8, w1b, b1t, k2, b2s)
    s_flat = s8.reshape(E)

    flat_idx = nbr_idx.reshape(E).astype(jnp.int32)
    psum, pcnt = _sc_scatter(flat_idx, s_flat)

    return _finish(psum, pcnt, atom_fea, tp_w)
